# baseline (speedup 1.0000x reference)
import jax
import jax.numpy as jnp
from jax import lax
from jax.experimental import pallas as pl
from jax.experimental.pallas import tpu as pltpu
from jax.experimental.pallas import tpu_sc as plsc
import functools

N, E, D, H, G = 10000, 320000, 128, 128, 64
NC, NS = 2, 16
NW = NC * NS
C = 80
NCHUNK = 128
EPW = NCHUNK * C
EPAD = NW * EPW
NP = N + 8
RPS = 624
TAIL = N - NS * RPS
NSLOT = 4
ISLOT = 8
ROUNDS = NCHUNK // NSLOT

BLK = 1000
NBLK = N // BLK


def _sc_agg_body(x_hbm, src_hbm, dst_hbm, zeros_hbm, out_hbm,
                 s0, s1, s2, s3, s4, s5, s6, s7,
                 d0, d1, d2, d3, d4, d5, d6, d7,
                 r0, r1, r2, r3,
                 i0, i1, i2, i3, i4, i5, i6, i7,
                 g0, g1, g2, g3,
                 t0, t1, t2, t3,
                 agg_sh):
    srcs = [s0, s1, s2, s3, s4, s5, s6, s7]
    dsts = [d0, d1, d2, d3, d4, d5, d6, d7]
    rows = [r0, r1, r2, r3]
    isems = [i0, i1, i2, i3, i4, i5, i6, i7]
    gsems = [g0, g1, g2, g3]
    ssems = [t0, t1, t2, t3]
    c = lax.axis_index("c")
    s = lax.axis_index("s")
    wid = c * NS + s

    pltpu.sync_copy(zeros_hbm.at[pl.ds(s * RPS, RPS)],
                    agg_sh.at[pl.ds(s * RPS, RPS)])

    @pl.when(s == NS - 1)
    def _():
        pltpu.sync_copy(zeros_hbm.at[pl.ds(NS * RPS, NP - NS * RPS)],
                        agg_sh.at[pl.ds(NS * RPS, NP - NS * RPS)])

    plsc.subcore_barrier()
    e_base = wid * EPW

    def idx_fetch(j, q):
        base = e_base + j * C
        pltpu.async_copy(src_hbm.at[pl.ds(base, C)], srcs[q], isems[q])
        pltpu.async_copy(dst_hbm.at[pl.ds(base, C)], dsts[q], isems[q])

    def idx_wait(q):
        pltpu.make_async_copy(src_hbm.at[pl.ds(0, C)], srcs[q], isems[q]).wait()
        pltpu.make_async_copy(dst_hbm.at[pl.ds(0, C)], dsts[q], isems[q]).wait()

    def gather(k, q):
        pltpu.async_copy(x_hbm.at[srcs[q]], rows[k], gsems[k])

    def gather_wait(k, q):
        pltpu.make_async_copy(x_hbm.at[srcs[q]], rows[k], gsems[k]).wait()

    def scat_start(k, q):
        pltpu.async_copy(rows[k], agg_sh.at[dsts[q]], ssems[k], add=True)

    def scat_wait(k, q):
        pltpu.make_async_copy(rows[k], agg_sh.at[dsts[q]], ssems[k]).wait()

    def do_round(jb, q, fetch, tail):
        for k in range(NSLOT):
            gather_wait(k, q + k)
            scat_start(k, q + k)
        for k in range(NSLOT):
            scat_wait(k, q + k)
            if fetch:
                idx_fetch(jb + 8 + k, q + k)
        if not tail:
            q2 = (q + 4) % ISLOT
            for k in range(NSLOT):
                idx_wait(q2 + k)
                gather(k, q2 + k)

    for q in range(ISLOT):
        idx_fetch(q, q)
    for k in range(NSLOT):
        idx_wait(k)
        gather(k, k)

    def body(i, carry):
        do_round(8 * i, 0, True, False)
        do_round(8 * i + 4, 4, True, False)
        return carry

    lax.fori_loop(0, ROUNDS // 2 - 1, body, 0)
    do_round(NCHUNK - 8, 0, False, False)
    do_round(NCHUNK - 4, 4, False, True)
    plsc.subcore_barrier()

    pltpu.sync_copy(agg_sh.at[pl.ds(s * RPS, RPS)],
                    out_hbm.at[c, pl.ds(s * RPS, RPS)])

    @pl.when(s == NS - 1)
    def _():
        pltpu.sync_copy(agg_sh.at[pl.ds(NS * RPS, TAIL)],
                        out_hbm.at[c, pl.ds(NS * RPS, TAIL)])


@functools.cache
def _sc_agg():
    return pl.kernel(
        _sc_agg_body,
        out_type=jax.ShapeDtypeStruct((NC, N, D), jnp.float32),
        mesh=plsc.VectorSubcoreMesh(core_axis_name="c", subcore_axis_name="s",
                                    num_cores=NC, num_subcores=NS),
        scratch_types=(
            [pltpu.VMEM((C,), jnp.int32)] * (2 * ISLOT)
            + [pltpu.VMEM((C, D), jnp.float32)] * NSLOT
            + [pltpu.SemaphoreType.DMA] * (ISLOT + 2 * NSLOT)
            + [pltpu.VMEM_SHARED((NP, D), jnp.float32)]
        ),
    )


def _tc_body(x_ref, parts_ref, batch_ref, W1_ref, b1_ref, W2_ref, b2_ref,
             W3_ref, b3_ref, out_ref, pooled_acc):
    i = pl.program_id(0)
    h = x_ref[...] + parts_ref[0] + parts_ref[1]
    h1 = jnp.dot(h, W1_ref[...], preferred_element_type=jnp.float32)
    h1 = jnp.maximum(h1 + b1_ref[...], 0.0)
    h2 = jnp.dot(h1, W2_ref[...], preferred_element_type=jnp.float32)
    h2 = h2 + b2_ref[...]
    bm = batch_ref[0, 0, :]
    gids = lax.broadcasted_iota(jnp.int32, (G, BLK), 0)
    mask = (bm[None, :] == gids).astype(jnp.float32)
    p = jnp.dot(mask, h2, preferred_element_type=jnp.float32)

    @pl.when(i == 0)
    def _():
        pooled_acc[...] = jnp.zeros_like(pooled_acc)

    pooled_acc[...] += p

    @pl.when(i == pl.num_programs(0) - 1)
    def _():
        out_ref[...] = (jnp.dot(pooled_acc[...], W3_ref[...],
                                preferred_element_type=jnp.float32)
                        + b3_ref[...])


@functools.partial(jax.jit)
def _tc_mlp_pool(x, parts, batch3, W1, b1, W2, b2, W3, b3):
    return pl.pallas_call(
        _tc_body,
        grid=(NBLK,),
        in_specs=[
            pl.BlockSpec((BLK, D), lambda i: (i, 0)),
            pl.BlockSpec((NC, BLK, D), lambda i: (0, i, 0)),
            pl.BlockSpec((1, 1, BLK), lambda i: (i, 0, 0)),
            pl.BlockSpec((D, H), lambda i: (0, 0)),
            pl.BlockSpec((1, H), lambda i: (0, 0)),
            pl.BlockSpec((H, H), lambda i: (0, 0)),
            pl.BlockSpec((1, H), lambda i: (0, 0)),
            pl.BlockSpec((H, 1), lambda i: (0, 0)),
            pl.BlockSpec((1, 1), lambda i: (0, 0)),
        ],
        out_specs=pl.BlockSpec((G, 1), lambda i: (0, 0)),
        out_shape=jax.ShapeDtypeStruct((G, 1), jnp.float32),
        scratch_shapes=[pltpu.VMEM((G, H), jnp.float32)],
        compiler_params=pltpu.CompilerParams(
            dimension_semantics=("arbitrary",)),
    )(x, parts, batch3, W1, b1, W2, b2, W3, b3)


def kernel(x, edge_index, batch, W1, b1, W2, b2, W3, b3):
    epw0 = E // NW
    pad = ((0, 0), (0, EPW - epw0))
    src = jnp.pad(edge_index[0].reshape(NW, epw0), pad,
                  constant_values=N).reshape(EPAD)
    dst = jnp.pad(edge_index[1].reshape(NW, epw0), pad,
                  constant_values=N).reshape(EPAD)
    x_pad = jnp.pad(x, ((0, NP - N), (0, 0)))
    zeros = jnp.zeros((NP, D), x.dtype)
    parts = _sc_agg()(x_pad, src, dst, zeros)
    out = _tc_mlp_pool(x, parts, batch.reshape(NBLK, 1, BLK),
                       W1, b1.reshape(1, H), W2, b2.reshape(1, H),
                       W3, b3.reshape(1, 1))
    return out


# speedup vs baseline: 1.0268x; 1.0268x over previous
import jax
import jax.numpy as jnp
from jax import lax
from jax.experimental import pallas as pl
from jax.experimental.pallas import tpu as pltpu
from jax.experimental.pallas import tpu_sc as plsc
import functools

N, E, D, H, G = 10000, 320000, 128, 128, 64
NC, NS = 2, 16
NW = NC * NS
C = 128
NCHUNK = 80
EPW = NCHUNK * C
EPAD = NW * EPW
NP = N + 8
RPS = 624
TAIL = N - NS * RPS

BLK = 1000
NBLK = N // BLK


def _sc_agg_body(x_hbm, src_hbm, dst_hbm, zeros_hbm, out_hbm,
                 src_a, src_b, dst_a, dst_b, rows_a, rows_b,
                 sem_a, sem_b, sem_ia, sem_ib, agg_sh):
    c = lax.axis_index("c")
    s = lax.axis_index("s")
    wid = c * NS + s

    pltpu.sync_copy(zeros_hbm.at[pl.ds(s * RPS, RPS)],
                    agg_sh.at[pl.ds(s * RPS, RPS)])

    @pl.when(s == NS - 1)
    def _():
        pltpu.sync_copy(zeros_hbm.at[pl.ds(NS * RPS, NP - NS * RPS)],
                        agg_sh.at[pl.ds(NS * RPS, NP - NS * RPS)])

    plsc.subcore_barrier()
    e_base = wid * EPW

    def idx_fetch(j, sbuf, dbuf, sem):
        base = e_base + j * C
        pltpu.async_copy(src_hbm.at[pl.ds(base, C)], sbuf, sem)
        pltpu.async_copy(dst_hbm.at[pl.ds(base, C)], dbuf, sem)

    def idx_wait(sbuf, dbuf, sem):
        pltpu.make_async_copy(src_hbm.at[pl.ds(0, C)], sbuf, sem).wait()
        pltpu.make_async_copy(dst_hbm.at[pl.ds(0, C)], dbuf, sem).wait()

    def gather(sbuf, buf, sem):
        pltpu.async_copy(x_hbm.at[sbuf], buf, sem)

    def gather_wait(sbuf, buf, sem):
        pltpu.make_async_copy(x_hbm.at[sbuf], buf, sem).wait()

    def scat(buf, dbuf):
        pltpu.sync_copy(buf, agg_sh.at[dbuf], add=True)

    idx_fetch(0, src_a, dst_a, sem_ia)
    idx_wait(src_a, dst_a, sem_ia)
    idx_fetch(1, src_b, dst_b, sem_ib)
    gather(src_a, rows_a, sem_a)

    def body(i, carry):
        ja = 2 * i
        jb = 2 * i + 1
        idx_wait(src_b, dst_b, sem_ib)
        gather_wait(src_a, rows_a, sem_a)
        gather(src_b, rows_b, sem_b)
        scat(rows_a, dst_a)
        idx_fetch(ja + 2, src_a, dst_a, sem_ia)
        idx_wait(src_a, dst_a, sem_ia)
        gather_wait(src_b, rows_b, sem_b)
        gather(src_a, rows_a, sem_a)
        scat(rows_b, dst_b)
        idx_fetch(jb + 2, src_b, dst_b, sem_ib)
        return carry

    lax.fori_loop(0, NCHUNK // 2 - 1, body, 0)
    idx_wait(src_b, dst_b, sem_ib)
    gather_wait(src_a, rows_a, sem_a)
    gather(src_b, rows_b, sem_b)
    scat(rows_a, dst_a)
    gather_wait(src_b, rows_b, sem_b)
    scat(rows_b, dst_b)
    plsc.subcore_barrier()

    pltpu.sync_copy(agg_sh.at[pl.ds(s * RPS, RPS)],
                    out_hbm.at[c, pl.ds(s * RPS, RPS)])

    @pl.when(s == NS - 1)
    def _():
        pltpu.sync_copy(agg_sh.at[pl.ds(NS * RPS, TAIL)],
                        out_hbm.at[c, pl.ds(NS * RPS, TAIL)])


@functools.cache
def _sc_agg():
    return pl.kernel(
        _sc_agg_body,
        out_type=jax.ShapeDtypeStruct((NC, N, D), jnp.float32),
        mesh=plsc.VectorSubcoreMesh(core_axis_name="c", subcore_axis_name="s",
                                    num_cores=NC, num_subcores=NS),
        scratch_types=[
            pltpu.VMEM((C,), jnp.int32),
            pltpu.VMEM((C,), jnp.int32),
            pltpu.VMEM((C,), jnp.int32),
            pltpu.VMEM((C,), jnp.int32),
            pltpu.VMEM((C, D), jnp.float32),
            pltpu.VMEM((C, D), jnp.float32),
            pltpu.SemaphoreType.DMA,
            pltpu.SemaphoreType.DMA,
            pltpu.SemaphoreType.DMA,
            pltpu.SemaphoreType.DMA,
            pltpu.VMEM_SHARED((NP, D), jnp.float32),
        ],
    )


def _tc_body(x_ref, parts_ref, batch_ref, W1_ref, b1_ref, W2_ref, b2_ref,
             W3_ref, b3_ref, out_ref, pooled_acc):
    i = pl.program_id(0)
    h = x_ref[...] + parts_ref[0] + parts_ref[1]
    h1 = jnp.dot(h, W1_ref[...], preferred_element_type=jnp.float32)
    h1 = jnp.maximum(h1 + b1_ref[...], 0.0)
    h2 = jnp.dot(h1, W2_ref[...], preferred_element_type=jnp.float32)
    h2 = h2 + b2_ref[...]
    bm = batch_ref[0, 0, :]
    gids = lax.broadcasted_iota(jnp.int32, (G, BLK), 0)
    mask = (bm[None, :] == gids).astype(jnp.float32)
    p = jnp.dot(mask, h2, preferred_element_type=jnp.float32)

    @pl.when(i == 0)
    def _():
        pooled_acc[...] = jnp.zeros_like(pooled_acc)

    pooled_acc[...] += p

    @pl.when(i == pl.num_programs(0) - 1)
    def _():
        out_ref[...] = (jnp.dot(pooled_acc[...], W3_ref[...],
                                preferred_element_type=jnp.float32)
                        + b3_ref[...])


@functools.partial(jax.jit)
def _tc_mlp_pool(x, parts, batch3, W1, b1, W2, b2, W3, b3):
    return pl.pallas_call(
        _tc_body,
        grid=(NBLK,),
        in_specs=[
            pl.BlockSpec((BLK, D), lambda i: (i, 0)),
            pl.BlockSpec((NC, BLK, D), lambda i: (0, i, 0)),
            pl.BlockSpec((1, 1, BLK), lambda i: (i, 0, 0)),
            pl.BlockSpec((D, H), lambda i: (0, 0)),
            pl.BlockSpec((1, H), lambda i: (0, 0)),
            pl.BlockSpec((H, H), lambda i: (0, 0)),
            pl.BlockSpec((1, H), lambda i: (0, 0)),
            pl.BlockSpec((H, 1), lambda i: (0, 0)),
            pl.BlockSpec((1, 1), lambda i: (0, 0)),
        ],
        out_specs=pl.BlockSpec((G, 1), lambda i: (0, 0)),
        out_shape=jax.ShapeDtypeStruct((G, 1), jnp.float32),
        scratch_shapes=[pltpu.VMEM((G, H), jnp.float32)],
        compiler_params=pltpu.CompilerParams(
            dimension_semantics=("arbitrary",)),
    )(x, parts, batch3, W1, b1, W2, b2, W3, b3)


def kernel(x, edge_index, batch, W1, b1, W2, b2, W3, b3):
    epw0 = E // NW
    pad = ((0, 0), (0, EPW - epw0))
    src = jnp.pad(edge_index[0].reshape(NW, epw0), pad,
                  constant_values=N).reshape(EPAD)
    dst = jnp.pad(edge_index[1].reshape(NW, epw0), pad,
                  constant_values=N).reshape(EPAD)
    x_pad = jnp.pad(x, ((0, NP - N), (0, 0)))
    zeros = jnp.zeros((NP, D), x.dtype)
    parts = _sc_agg()(x_pad, src, dst, zeros)
    out = _tc_mlp_pool(x, parts, batch.reshape(NBLK, 1, BLK),
                       W1, b1.reshape(1, H), W2, b2.reshape(1, H),
                       W3, b3.reshape(1, 1))
    return out


# speedup vs baseline: 2.3298x; 2.2690x over previous
import jax
import jax.numpy as jnp
from jax import lax
from jax.experimental import pallas as pl
from jax.experimental.pallas import tpu as pltpu
from jax.experimental.pallas import tpu_sc as plsc
import functools

N, E, D, H, G = 10000, 320000, 128, 128, 64
NC, NS = 2, 16
NW = NC * NS
C = 128
NCHUNK = 80
EPW = NCHUNK * C
EPAD = NW * EPW
NP = N + 40
RPS = 624
TAIL = N - NS * RPS

BLK = 1000
NBLK = N // BLK


def _sc_agg_body(x_hbm, src_hbm, dst_hbm, zeros_hbm, out_hbm,
                 src_a, src_b, dst_a, dst_b, rows_a, rows_b,
                 sem_a, sem_b, sem_ia, sem_ib, agg_sh):
    c = lax.axis_index("c")
    s = lax.axis_index("s")
    wid = c * NS + s

    pltpu.sync_copy(zeros_hbm.at[pl.ds(s * RPS, RPS)],
                    agg_sh.at[pl.ds(s * RPS, RPS)])

    @pl.when(s == NS - 1)
    def _():
        pltpu.sync_copy(zeros_hbm.at[pl.ds(NS * RPS, NP - NS * RPS)],
                        agg_sh.at[pl.ds(NS * RPS, NP - NS * RPS)])

    plsc.subcore_barrier()
    e_base = wid * EPW

    def idx_fetch(j, sbuf, dbuf, sem):
        base = e_base + j * C
        pltpu.async_copy(src_hbm.at[pl.ds(base, C)], sbuf, sem)
        pltpu.async_copy(dst_hbm.at[pl.ds(base, C)], dbuf, sem)

    def idx_wait(sbuf, dbuf, sem):
        pltpu.make_async_copy(src_hbm.at[pl.ds(0, C)], sbuf, sem).wait()
        pltpu.make_async_copy(dst_hbm.at[pl.ds(0, C)], dbuf, sem).wait()

    def gather(sbuf, buf, sem):
        pltpu.async_copy(x_hbm.at[sbuf], buf, sem)

    def gather_wait(sbuf, buf, sem):
        pltpu.make_async_copy(x_hbm.at[sbuf], buf, sem).wait()

    def scat(buf, dbuf):
        pltpu.sync_copy(buf, agg_sh.at[dbuf], add=True)

    idx_fetch(0, src_a, dst_a, sem_ia)
    idx_wait(src_a, dst_a, sem_ia)
    idx_fetch(1, src_b, dst_b, sem_ib)
    gather(src_a, rows_a, sem_a)

    def body(i, carry):
        ja = 2 * i
        jb = 2 * i + 1
        idx_wait(src_b, dst_b, sem_ib)
        gather_wait(src_a, rows_a, sem_a)
        gather(src_b, rows_b, sem_b)
        scat(rows_a, dst_a)
        idx_fetch(ja + 2, src_a, dst_a, sem_ia)
        idx_wait(src_a, dst_a, sem_ia)
        gather_wait(src_b, rows_b, sem_b)
        gather(src_a, rows_a, sem_a)
        scat(rows_b, dst_b)
        idx_fetch(jb + 2, src_b, dst_b, sem_ib)
        return carry

    lax.fori_loop(0, NCHUNK // 2 - 1, body, 0)
    idx_wait(src_b, dst_b, sem_ib)
    gather_wait(src_a, rows_a, sem_a)
    gather(src_b, rows_b, sem_b)
    scat(rows_a, dst_a)
    gather_wait(src_b, rows_b, sem_b)
    scat(rows_b, dst_b)
    plsc.subcore_barrier()

    pltpu.sync_copy(agg_sh.at[pl.ds(s * RPS, RPS)],
                    out_hbm.at[c, pl.ds(s * RPS, RPS)])

    @pl.when(s == NS - 1)
    def _():
        pltpu.sync_copy(agg_sh.at[pl.ds(NS * RPS, TAIL)],
                        out_hbm.at[c, pl.ds(NS * RPS, TAIL)])


@functools.cache
def _sc_agg():
    return pl.kernel(
        _sc_agg_body,
        out_type=jax.ShapeDtypeStruct((NC, N, D), jnp.float32),
        mesh=plsc.VectorSubcoreMesh(core_axis_name="c", subcore_axis_name="s",
                                    num_cores=NC, num_subcores=NS),
        scratch_types=[
            pltpu.VMEM((C,), jnp.int32),
            pltpu.VMEM((C,), jnp.int32),
            pltpu.VMEM((C,), jnp.int32),
            pltpu.VMEM((C,), jnp.int32),
            pltpu.VMEM((C, D), jnp.float32),
            pltpu.VMEM((C, D), jnp.float32),
            pltpu.SemaphoreType.DMA,
            pltpu.SemaphoreType.DMA,
            pltpu.SemaphoreType.DMA,
            pltpu.SemaphoreType.DMA,
            pltpu.VMEM_SHARED((NP, D), jnp.float32),
        ],
    )


def _tc_body(x_ref, parts_ref, batch_ref, W1_ref, b1_ref, W2_ref, b2_ref,
             W3_ref, b3_ref, out_ref, pooled_acc):
    i = pl.program_id(0)
    h = x_ref[...] + parts_ref[0] + parts_ref[1]
    h1 = jnp.dot(h, W1_ref[...], preferred_element_type=jnp.float32)
    h1 = jnp.maximum(h1 + b1_ref[...], 0.0)
    h2 = jnp.dot(h1, W2_ref[...], preferred_element_type=jnp.float32)
    h2 = h2 + b2_ref[...]
    bm = batch_ref[0, 0, :]
    gids = lax.broadcasted_iota(jnp.int32, (G, BLK), 0)
    mask = (bm[None, :] == gids).astype(jnp.float32)
    p = jnp.dot(mask, h2, preferred_element_type=jnp.float32)

    @pl.when(i == 0)
    def _():
        pooled_acc[...] = jnp.zeros_like(pooled_acc)

    pooled_acc[...] += p

    @pl.when(i == pl.num_programs(0) - 1)
    def _():
        out_ref[...] = (jnp.dot(pooled_acc[...], W3_ref[...],
                                preferred_element_type=jnp.float32)
                        + b3_ref[...])


@functools.partial(jax.jit)
def _tc_mlp_pool(x, parts, batch3, W1, b1, W2, b2, W3, b3):
    return pl.pallas_call(
        _tc_body,
        grid=(NBLK,),
        in_specs=[
            pl.BlockSpec((BLK, D), lambda i: (i, 0)),
            pl.BlockSpec((NC, BLK, D), lambda i: (0, i, 0)),
            pl.BlockSpec((1, 1, BLK), lambda i: (i, 0, 0)),
            pl.BlockSpec((D, H), lambda i: (0, 0)),
            pl.BlockSpec((1, H), lambda i: (0, 0)),
            pl.BlockSpec((H, H), lambda i: (0, 0)),
            pl.BlockSpec((1, H), lambda i: (0, 0)),
            pl.BlockSpec((H, 1), lambda i: (0, 0)),
            pl.BlockSpec((1, 1), lambda i: (0, 0)),
        ],
        out_specs=pl.BlockSpec((G, 1), lambda i: (0, 0)),
        out_shape=jax.ShapeDtypeStruct((G, 1), jnp.float32),
        scratch_shapes=[pltpu.VMEM((G, H), jnp.float32)],
        compiler_params=pltpu.CompilerParams(
            dimension_semantics=("arbitrary",)),
    )(x, parts, batch3, W1, b1, W2, b2, W3, b3)


def kernel(x, edge_index, batch, W1, b1, W2, b2, W3, b3):
    epw0 = E // NW
    padv = jnp.broadcast_to(
        (N + jnp.arange(NW, dtype=jnp.int32))[:, None], (NW, EPW - epw0))
    src = jnp.concatenate(
        [edge_index[0].reshape(NW, epw0), padv], axis=1).reshape(EPAD)
    dst = jnp.concatenate(
        [edge_index[1].reshape(NW, epw0), padv], axis=1).reshape(EPAD)
    x_pad = jnp.pad(x, ((0, NP - N), (0, 0)))
    zeros = jnp.zeros((NP, D), x.dtype)
    parts = _sc_agg()(x_pad, src, dst, zeros)
    out = _tc_mlp_pool(x, parts, batch.reshape(NBLK, 1, BLK),
                       W1, b1.reshape(1, H), W2, b2.reshape(1, H),
                       W3, b3.reshape(1, 1))
    return out


# speedup vs baseline: 2.4183x; 1.0380x over previous
import jax
import jax.numpy as jnp
from jax import lax
from jax.experimental import pallas as pl
from jax.experimental.pallas import tpu as pltpu
from jax.experimental.pallas import tpu_sc as plsc
import functools

N, E, D, H, G = 10000, 320000, 128, 128, 64
NC, NS = 2, 16
NW = NC * NS
EPW = E // NW
C = 80
NCHUNK = EPW // C
RPS = 624
TAIL = N - NS * RPS

BLK = 1000
NBLK = N // BLK


def _sc_agg_body(x_hbm, src_hbm, dst_hbm, zeros_hbm, out_hbm,
                 s0, s1, s2, s3, d0, d1, d2, d3, rows_a, rows_b,
                 i0, i1, i2, i3, sem_a, sem_b, agg_sh):
    srcs = [s0, s1, s2, s3]
    dsts = [d0, d1, d2, d3]
    rows = [rows_a, rows_b]
    isems = [i0, i1, i2, i3]
    gsems = [sem_a, sem_b]
    c = lax.axis_index("c")
    s = lax.axis_index("s")
    wid = c * NS + s

    @pl.when(c == 0)
    def _():
        pltpu.sync_copy(x_hbm.at[pl.ds(s * RPS, RPS)],
                        agg_sh.at[pl.ds(s * RPS, RPS)])

        @pl.when(s == NS - 1)
        def _():
            pltpu.sync_copy(x_hbm.at[pl.ds(NS * RPS, TAIL)],
                            agg_sh.at[pl.ds(NS * RPS, TAIL)])

    @pl.when(c == 1)
    def _():
        pltpu.sync_copy(zeros_hbm.at[pl.ds(s * RPS, RPS)],
                        agg_sh.at[pl.ds(s * RPS, RPS)])

        @pl.when(s == NS - 1)
        def _():
            pltpu.sync_copy(zeros_hbm.at[pl.ds(NS * RPS, TAIL)],
                            agg_sh.at[pl.ds(NS * RPS, TAIL)])

    plsc.subcore_barrier()
    e_base = wid * EPW

    def idx_fetch(j, q):
        base = e_base + j * C
        pltpu.async_copy(src_hbm.at[pl.ds(base, C)], srcs[q], isems[q])
        pltpu.async_copy(dst_hbm.at[pl.ds(base, C)], dsts[q], isems[q])

    def idx_wait(q):
        pltpu.make_async_copy(src_hbm.at[pl.ds(0, C)], srcs[q], isems[q]).wait()
        pltpu.make_async_copy(dst_hbm.at[pl.ds(0, C)], dsts[q], isems[q]).wait()

    def gather(q, k):
        pltpu.async_copy(x_hbm.at[srcs[q]], rows[k], gsems[k])

    def gather_wait(q, k):
        pltpu.make_async_copy(x_hbm.at[srcs[q]], rows[k], gsems[k]).wait()

    def scat(k, q):
        pltpu.sync_copy(rows[k], agg_sh.at[dsts[q]], add=True)

    def phase(j, m, fetch):
        qn, kn = (m + 1) % 4, (m + 1) % 2
        idx_wait(qn)
        gather_wait(m, m % 2)
        gather(qn, kn)
        scat(m % 2, m)
        if fetch:
            idx_fetch(j + 4, m)

    for q in range(4):
        idx_fetch(q, q)
    idx_wait(0)
    gather(0, 0)

    def body(i, carry):
        for m in range(4):
            phase(4 * i + m, m, True)
        return carry

    lax.fori_loop(0, NCHUNK // 4 - 1, body, 0)
    for m in range(4):
        phase(NCHUNK - 5 + m, m, m == 0)
    gather_wait(0, 0)
    scat(0, 0)
    plsc.subcore_barrier()

    pltpu.sync_copy(agg_sh.at[pl.ds(s * RPS, RPS)],
                    out_hbm.at[c, pl.ds(s * RPS, RPS)])

    @pl.when(s == NS - 1)
    def _():
        pltpu.sync_copy(agg_sh.at[pl.ds(NS * RPS, TAIL)],
                        out_hbm.at[c, pl.ds(NS * RPS, TAIL)])


@functools.cache
def _sc_agg():
    return pl.kernel(
        _sc_agg_body,
        out_type=jax.ShapeDtypeStruct((NC, N, D), jnp.float32),
        mesh=plsc.VectorSubcoreMesh(core_axis_name="c", subcore_axis_name="s",
                                    num_cores=NC, num_subcores=NS),
        scratch_types=(
            [pltpu.VMEM((C,), jnp.int32)] * 8
            + [pltpu.VMEM((C, D), jnp.float32)] * 2
            + [pltpu.SemaphoreType.DMA] * 6
            + [pltpu.VMEM_SHARED((N, D), jnp.float32)]
        ),
    )


def _tc_body(parts_ref, batch_ref, W1_ref, b1_ref, W2_ref, b2_ref,
             W3_ref, b3_ref, out_ref, pooled_acc):
    i = pl.program_id(0)
    h = parts_ref[0] + parts_ref[1]
    h1 = jnp.dot(h, W1_ref[...], preferred_element_type=jnp.float32)
    h1 = jnp.maximum(h1 + b1_ref[...], 0.0)
    h2 = jnp.dot(h1, W2_ref[...], preferred_element_type=jnp.float32)
    h2 = h2 + b2_ref[...]
    bm = batch_ref[0, 0, :]
    gids = lax.broadcasted_iota(jnp.int32, (G, BLK), 0)
    mask = (bm[None, :] == gids).astype(jnp.float32)
    p = jnp.dot(mask, h2, preferred_element_type=jnp.float32)

    @pl.when(i == 0)
    def _():
        pooled_acc[...] = jnp.zeros_like(pooled_acc)

    pooled_acc[...] += p

    @pl.when(i == pl.num_programs(0) - 1)
    def _():
        out_ref[...] = (jnp.dot(pooled_acc[...], W3_ref[...],
                                preferred_element_type=jnp.float32)
                        + b3_ref[...])


@functools.partial(jax.jit)
def _tc_mlp_pool(parts, batch3, W1, b1, W2, b2, W3, b3):
    return pl.pallas_call(
        _tc_body,
        grid=(NBLK,),
        in_specs=[
            pl.BlockSpec((NC, BLK, D), lambda i: (0, i, 0)),
            pl.BlockSpec((1, 1, BLK), lambda i: (i, 0, 0)),
            pl.BlockSpec((D, H), lambda i: (0, 0)),
            pl.BlockSpec((1, H), lambda i: (0, 0)),
            pl.BlockSpec((H, H), lambda i: (0, 0)),
            pl.BlockSpec((1, H), lambda i: (0, 0)),
            pl.BlockSpec((H, 1), lambda i: (0, 0)),
            pl.BlockSpec((1, 1), lambda i: (0, 0)),
        ],
        out_specs=pl.BlockSpec((G, 1), lambda i: (0, 0)),
        out_shape=jax.ShapeDtypeStruct((G, 1), jnp.float32),
        scratch_shapes=[pltpu.VMEM((G, H), jnp.float32)],
        compiler_params=pltpu.CompilerParams(
            dimension_semantics=("arbitrary",)),
    )(parts, batch3, W1, b1, W2, b2, W3, b3)


def kernel(x, edge_index, batch, W1, b1, W2, b2, W3, b3):
    src = edge_index[0]
    dst = edge_index[1]
    zeros = jnp.zeros_like(x)
    parts = _sc_agg()(x, src, dst, zeros)
    out = _tc_mlp_pool(parts, batch.reshape(NBLK, 1, BLK),
                       W1, b1.reshape(1, H), W2, b2.reshape(1, H),
                       W3, b3.reshape(1, 1))
    return out


# speedup vs baseline: 2.4788x; 1.0250x over previous
import jax
import jax.numpy as jnp
from jax import lax
from jax.experimental import pallas as pl
from jax.experimental.pallas import tpu as pltpu
from jax.experimental.pallas import tpu_sc as plsc
import functools

N, E, D, H, G = 10000, 320000, 128, 128, 64
NC, NS = 2, 16
NW = NC * NS
EPW = E // NW
C = 80
NCHUNK = EPW // C
RPS = 624
TAIL = N - NS * RPS

BLK = 2000
NBLK = N // BLK


def _sc_agg_body(x_hbm, src_hbm, dst_hbm, zeros_hbm, out_hbm,
                 s0, s1, s2, s3, d0, d1, d2, d3, rows_a, rows_b,
                 i0, i1, i2, i3, sem_a, sem_b, agg_sh):
    srcs = [s0, s1, s2, s3]
    dsts = [d0, d1, d2, d3]
    rows = [rows_a, rows_b]
    isems = [i0, i1, i2, i3]
    gsems = [sem_a, sem_b]
    c = lax.axis_index("c")
    s = lax.axis_index("s")
    wid = c * NS + s

    @pl.when(c == 0)
    def _():
        pltpu.sync_copy(x_hbm.at[pl.ds(s * RPS, RPS)],
                        agg_sh.at[pl.ds(s * RPS, RPS)])

        @pl.when(s == NS - 1)
        def _():
            pltpu.sync_copy(x_hbm.at[pl.ds(NS * RPS, TAIL)],
                            agg_sh.at[pl.ds(NS * RPS, TAIL)])

    @pl.when(c == 1)
    def _():
        pltpu.sync_copy(zeros_hbm, agg_sh.at[pl.ds(s * RPS, RPS)])

        @pl.when(s == NS - 1)
        def _():
            pltpu.sync_copy(zeros_hbm.at[pl.ds(0, TAIL)],
                            agg_sh.at[pl.ds(NS * RPS, TAIL)])

    plsc.subcore_barrier()
    e_base = wid * EPW

    def idx_fetch(j, q):
        base = e_base + j * C
        pltpu.async_copy(src_hbm.at[pl.ds(base, C)], srcs[q], isems[q])
        pltpu.async_copy(dst_hbm.at[pl.ds(base, C)], dsts[q], isems[q])

    def idx_wait(q):
        pltpu.make_async_copy(src_hbm.at[pl.ds(0, C)], srcs[q], isems[q]).wait()
        pltpu.make_async_copy(dst_hbm.at[pl.ds(0, C)], dsts[q], isems[q]).wait()

    def gather(q, k):
        pltpu.async_copy(x_hbm.at[srcs[q]], rows[k], gsems[k])

    def gather_wait(q, k):
        pltpu.make_async_copy(x_hbm.at[srcs[q]], rows[k], gsems[k]).wait()

    def scat(k, q):
        pltpu.sync_copy(rows[k], agg_sh.at[dsts[q]], add=True)

    def phase(j, m, fetch):
        qn, kn = (m + 1) % 4, (m + 1) % 2
        idx_wait(qn)
        gather_wait(m, m % 2)
        gather(qn, kn)
        scat(m % 2, m)
        if fetch:
            idx_fetch(j + 4, m)

    for q in range(4):
        idx_fetch(q, q)
    idx_wait(0)
    gather(0, 0)

    def body(i, carry):
        for m in range(4):
            phase(4 * i + m, m, True)
        return carry

    lax.fori_loop(0, NCHUNK // 4 - 1, body, 0)
    for m in range(4):
        phase(NCHUNK - 5 + m, m, m == 0)
    gather_wait(0, 0)
    scat(0, 0)
    plsc.subcore_barrier()

    pltpu.sync_copy(agg_sh.at[pl.ds(s * RPS, RPS)],
                    out_hbm.at[c, pl.ds(s * RPS, RPS)])

    @pl.when(s == NS - 1)
    def _():
        pltpu.sync_copy(agg_sh.at[pl.ds(NS * RPS, TAIL)],
                        out_hbm.at[c, pl.ds(NS * RPS, TAIL)])


@functools.cache
def _sc_agg():
    return pl.kernel(
        _sc_agg_body,
        out_type=jax.ShapeDtypeStruct((NC, N, D), jnp.float32),
        mesh=plsc.VectorSubcoreMesh(core_axis_name="c", subcore_axis_name="s",
                                    num_cores=NC, num_subcores=NS),
        scratch_types=(
            [pltpu.VMEM((C,), jnp.int32)] * 8
            + [pltpu.VMEM((C, D), jnp.float32)] * 2
            + [pltpu.SemaphoreType.DMA] * 6
            + [pltpu.VMEM_SHARED((N, D), jnp.float32)]
        ),
    )


def _tc_body(parts_ref, batch_ref, W1_ref, b1_ref, W2_ref, b2_ref,
             W3_ref, b3_ref, out_ref, pooled_acc):
    i = pl.program_id(0)
    h = parts_ref[0] + parts_ref[1]
    h1 = jnp.dot(h, W1_ref[...], preferred_element_type=jnp.float32)
    h1 = jnp.maximum(h1 + b1_ref[...], 0.0)
    h2 = jnp.dot(h1, W2_ref[...], preferred_element_type=jnp.float32)
    h2 = h2 + b2_ref[...]
    bm = batch_ref[0, 0, :]
    gids = lax.broadcasted_iota(jnp.int32, (G, BLK), 0)
    mask = (bm[None, :] == gids).astype(jnp.float32)
    p = jnp.dot(mask, h2, preferred_element_type=jnp.float32)

    @pl.when(i == 0)
    def _():
        pooled_acc[...] = jnp.zeros_like(pooled_acc)

    pooled_acc[...] += p

    @pl.when(i == pl.num_programs(0) - 1)
    def _():
        out_ref[...] = (jnp.dot(pooled_acc[...], W3_ref[...],
                                preferred_element_type=jnp.float32)
                        + b3_ref[...])


@functools.partial(jax.jit)
def _tc_mlp_pool(parts, batch3, W1, b1, W2, b2, W3, b3):
    return pl.pallas_call(
        _tc_body,
        grid=(NBLK,),
        in_specs=[
            pl.BlockSpec((NC, BLK, D), lambda i: (0, i, 0)),
            pl.BlockSpec((1, 1, BLK), lambda i: (i, 0, 0)),
            pl.BlockSpec((D, H), lambda i: (0, 0)),
            pl.BlockSpec((1, H), lambda i: (0, 0)),
            pl.BlockSpec((H, H), lambda i: (0, 0)),
            pl.BlockSpec((1, H), lambda i: (0, 0)),
            pl.BlockSpec((H, 1), lambda i: (0, 0)),
            pl.BlockSpec((1, 1), lambda i: (0, 0)),
        ],
        out_specs=pl.BlockSpec((G, 1), lambda i: (0, 0)),
        out_shape=jax.ShapeDtypeStruct((G, 1), jnp.float32),
        scratch_shapes=[pltpu.VMEM((G, H), jnp.float32)],
        compiler_params=pltpu.CompilerParams(
            dimension_semantics=("arbitrary",)),
    )(parts, batch3, W1, b1, W2, b2, W3, b3)


def kernel(x, edge_index, batch, W1, b1, W2, b2, W3, b3):
    src = edge_index[0]
    dst = edge_index[1]
    zeros = jnp.zeros((RPS, D), x.dtype)
    parts = _sc_agg()(x, src, dst, zeros)
    out = _tc_mlp_pool(parts, batch.reshape(NBLK, 1, BLK),
                       W1, b1.reshape(1, H), W2, b2.reshape(1, H),
                       W3, b3.reshape(1, 1))
    return out


# speedup vs baseline: 2.7554x; 1.1116x over previous
import jax
import jax.numpy as jnp
from jax import lax
from jax.experimental import pallas as pl
from jax.experimental.pallas import tpu as pltpu
from jax.experimental.pallas import tpu_sc as plsc
import functools

N, E, D, H, G = 10000, 320000, 128, 128, 64
NC, NS = 2, 16
NW = NC * NS
EPW = E // NW
C = 80
NCHUNK = EPW // C
RPS = 624
TAIL = N - NS * RPS

BLK = 2000
NBLK = N // BLK


def _sc_agg_body(x_hbm, src_hbm, dst_hbm, zeros_hbm, out_hbm,
                 s0, s1, s2, d0, d1, d2, r0, r1, r2,
                 i0, i1, i2, g0, g1, g2, agg_sh):
    srcs = [s0, s1, s2]
    dsts = [d0, d1, d2]
    rows = [r0, r1, r2]
    isems = [i0, i1, i2]
    gsems = [g0, g1, g2]
    c = lax.axis_index("c")
    s = lax.axis_index("s")
    wid = c * NS + s

    @pl.when(c == 0)
    def _():
        pltpu.sync_copy(x_hbm.at[pl.ds(s * RPS, RPS)],
                        agg_sh.at[pl.ds(s * RPS, RPS)])

        @pl.when(s == NS - 1)
        def _():
            pltpu.sync_copy(x_hbm.at[pl.ds(NS * RPS, TAIL)],
                            agg_sh.at[pl.ds(NS * RPS, TAIL)])

    @pl.when(c == 1)
    def _():
        pltpu.sync_copy(zeros_hbm, agg_sh.at[pl.ds(s * RPS, RPS)])

        @pl.when(s == NS - 1)
        def _():
            pltpu.sync_copy(zeros_hbm.at[pl.ds(0, TAIL)],
                            agg_sh.at[pl.ds(NS * RPS, TAIL)])

    plsc.subcore_barrier()
    e_base = wid * EPW

    def idx_fetch(j, q):
        base = e_base + j * C
        pltpu.async_copy(src_hbm.at[pl.ds(base, C)], srcs[q], isems[q])
        pltpu.async_copy(dst_hbm.at[pl.ds(base, C)], dsts[q], isems[q])

    def idx_wait(q):
        pltpu.make_async_copy(src_hbm.at[pl.ds(0, C)], srcs[q], isems[q]).wait()
        pltpu.make_async_copy(dst_hbm.at[pl.ds(0, C)], dsts[q], isems[q]).wait()

    def gather(q, k):
        pltpu.async_copy(x_hbm.at[srcs[q]], rows[k], gsems[k])

    def gather_wait(q, k):
        pltpu.make_async_copy(x_hbm.at[srcs[q]], rows[k], gsems[k]).wait()

    def scat(k, q):
        pltpu.sync_copy(rows[k], agg_sh.at[dsts[q]], add=True)

    def phase(j, m, fetch, issue):
        m1, m2 = (m + 1) % 3, (m + 2) % 3
        if issue:
            idx_wait(m2)
        gather_wait(m, m)
        if issue:
            gather(m2, m2)
        scat(m, m)
        if fetch:
            idx_fetch(j + 3, m)

    for q in range(3):
        idx_fetch(q, q)
    idx_wait(0)
    idx_wait(1)
    gather(0, 0)
    gather(1, 1)

    def body(i, carry):
        for m in range(3):
            phase(3 * i + m, m, True, True)
        return carry

    lax.fori_loop(0, 40, body, 0)
    phase(120, 0, True, True)
    phase(121, 1, True, True)
    phase(122, 2, False, True)
    phase(123, 0, False, False)
    phase(124, 1, False, False)
    plsc.subcore_barrier()

    pltpu.sync_copy(agg_sh.at[pl.ds(s * RPS, RPS)],
                    out_hbm.at[c, pl.ds(s * RPS, RPS)])

    @pl.when(s == NS - 1)
    def _():
        pltpu.sync_copy(agg_sh.at[pl.ds(NS * RPS, TAIL)],
                        out_hbm.at[c, pl.ds(NS * RPS, TAIL)])


@functools.cache
def _sc_agg():
    return pl.kernel(
        _sc_agg_body,
        out_type=jax.ShapeDtypeStruct((NC, N, D), jnp.float32),
        mesh=plsc.VectorSubcoreMesh(core_axis_name="c", subcore_axis_name="s",
                                    num_cores=NC, num_subcores=NS),
        scratch_types=(
            [pltpu.VMEM((C,), jnp.int32)] * 6
            + [pltpu.VMEM((C, D), jnp.float32)] * 3
            + [pltpu.SemaphoreType.DMA] * 6
            + [pltpu.VMEM_SHARED((N, D), jnp.float32)]
        ),
    )


def _tc_body(parts_ref, batch_ref, W1_ref, b1_ref, W2_ref, b2_ref,
             W3_ref, b3_ref, out_ref, pooled_acc):
    i = pl.program_id(0)
    h = parts_ref[0] + parts_ref[1]
    h1 = jnp.dot(h, W1_ref[...], preferred_element_type=jnp.float32)
    h1 = jnp.maximum(h1 + b1_ref[...], 0.0)
    h2 = jnp.dot(h1, W2_ref[...], preferred_element_type=jnp.float32)
    h2 = h2 + b2_ref[...]
    bm = batch_ref[0, 0, :]
    gids = lax.broadcasted_iota(jnp.int32, (G, BLK), 0)
    mask = (bm[None, :] == gids).astype(jnp.float32)
    p = jnp.dot(mask, h2, preferred_element_type=jnp.float32)

    @pl.when(i == 0)
    def _():
        pooled_acc[...] = jnp.zeros_like(pooled_acc)

    pooled_acc[...] += p

    @pl.when(i == pl.num_programs(0) - 1)
    def _():
        out_ref[...] = (jnp.dot(pooled_acc[...], W3_ref[...],
                                preferred_element_type=jnp.float32)
                        + b3_ref[...])


@functools.partial(jax.jit)
def _tc_mlp_pool(parts, batch3, W1, b1, W2, b2, W3, b3):
    return pl.pallas_call(
        _tc_body,
        grid=(NBLK,),
        in_specs=[
            pl.BlockSpec((NC, BLK, D), lambda i: (0, i, 0)),
            pl.BlockSpec((1, 1, BLK), lambda i: (i, 0, 0)),
            pl.BlockSpec((D, H), lambda i: (0, 0)),
            pl.BlockSpec((1, H), lambda i: (0, 0)),
            pl.BlockSpec((H, H), lambda i: (0, 0)),
            pl.BlockSpec((1, H), lambda i: (0, 0)),
            pl.BlockSpec((H, 1), lambda i: (0, 0)),
            pl.BlockSpec((1, 1), lambda i: (0, 0)),
        ],
        out_specs=pl.BlockSpec((G, 1), lambda i: (0, 0)),
        out_shape=jax.ShapeDtypeStruct((G, 1), jnp.float32),
        scratch_shapes=[pltpu.VMEM((G, H), jnp.float32)],
        compiler_params=pltpu.CompilerParams(
            dimension_semantics=("arbitrary",)),
    )(parts, batch3, W1, b1, W2, b2, W3, b3)


def kernel(x, edge_index, batch, W1, b1, W2, b2, W3, b3):
    src = edge_index[0]
    dst = edge_index[1]
    zeros = jnp.zeros((RPS, D), x.dtype)
    parts = _sc_agg()(x, src, dst, zeros)
    out = _tc_mlp_pool(parts, batch.reshape(NBLK, 1, BLK),
                       W1, b1.reshape(1, H), W2, b2.reshape(1, H),
                       W3, b3.reshape(1, 1))
    return out
